# Initial kernel scaffold; baseline (speedup 1.0000x reference)
import jax
import jax.numpy as jnp
from jax.experimental import pallas as pl


def kernel(x, e, first, second, segment, l_w1, l_b1, l_w2, l_b2, b_w1, b_b1, b_w2, b_b2, gru_k, gru_rk, gru_b, i_w1, i_b1, i_w2, i_b2, j_w1, j_b1, j_w2, j_b2, f_w1, f_b1, f_w2, f_b2):
    raise NotImplementedError("write your pallas kernel here")


# speedup vs baseline: 2.1821x; 2.1821x over previous
import functools

import jax
import jax.numpy as jnp
from jax import lax
from jax.experimental import pallas as pl
from jax.experimental.pallas import tpu as pltpu
from jax.experimental.pallas import tpu_sc as plsc

N = 50000
E = 800000
N_H = 8
G = 128

NC = 2
NS = 16
EW = E // (NC * NS)
C = 125
NCH = EW // C
NROW = N // NS

EBLK = 8000
NEB = E // EBLK
NBLK = 2000
NNB = N // NBLK


def _selu(v):
    return 1.0507009873554805 * jnp.where(
        v > 0, v, 1.6732632423543772 * (jnp.exp(jnp.minimum(v, 0.0)) - 1.0))


def _edge_mlp_body(e_ref, lw1, lb1, lw2, lb2, bw1, bb1, bw2, bb2, a_ref, bb_ref):
    e = e_ref[...]
    s1 = _selu(e * lw1[...] + lb1[...])
    a_ref[...] = jnp.dot(s1, lw2[...], preferred_element_type=jnp.float32) + lb2[...]
    s2 = _selu(e * bw1[...] + bb1[...])
    bb_ref[...] = jnp.dot(s2, bw2[...], preferred_element_type=jnp.float32) + bb2[...]


def _edge_mlps(e, l_w1, l_b1, l_w2, l_b2, b_w1, b_b1, b_w2, b_b2):
    full = lambda shp: pl.BlockSpec(shp, lambda i: (0,) * len(shp))
    return pl.pallas_call(
        _edge_mlp_body,
        grid=(NEB,),
        in_specs=[
            pl.BlockSpec((EBLK, 1), lambda i: (i, 0)),
            full((1, 64)), full((1, 64)), full((64, 64)), full((1, 64)),
            full((1, 64)), full((1, 64)), full((64, 8)), full((1, 8)),
        ],
        out_specs=[
            pl.BlockSpec((EBLK, 64), lambda i: (i, 0)),
            pl.BlockSpec((EBLK, 8), lambda i: (i, 0)),
        ],
        out_shape=[
            jax.ShapeDtypeStruct((E, 64), jnp.float32),
            jax.ShapeDtypeStruct((E, 8), jnp.float32),
        ],
    )(e, l_w1, l_b1.reshape(1, 64), l_w2, l_b2.reshape(1, 64),
      b_w1, b_b1.reshape(1, 64), b_w2, b_b2.reshape(1, 8))


def _gather_body(h_hbm, idx_hbm, out_hbm, idx_v, rows_v, sem):
    c = lax.axis_index("c")
    s = lax.axis_index("s")
    pltpu.sync_copy(idx_hbm.at[c, s], idx_v)

    @pl.loop(0, NCH)
    def _chunk(ch):
        pltpu.async_copy(h_hbm.at[idx_v.at[ch]], rows_v, sem).wait()
        pltpu.sync_copy(rows_v, out_hbm.at[c, s, ch])


@functools.cache
def _sc_gather_kernel():
    return pl.kernel(
        _gather_body,
        out_type=jax.ShapeDtypeStruct((NC, NS, NCH, C, N_H), jnp.float32),
        mesh=plsc.VectorSubcoreMesh(core_axis_name="c", subcore_axis_name="s",
                                    num_cores=NC, num_subcores=NS),
        scratch_types=[
            pltpu.VMEM((NCH, C), jnp.int32),
            pltpu.VMEM((C, N_H), jnp.float32),
            pltpu.SemaphoreType.DMA,
        ],
        compiler_params=pltpu.CompilerParams(use_tc_tiling_on_sc=False),
    )


def _scatter_body(m_hbm, idx_hbm, zeros_hbm, out_hbm, m_v, idx_v, acc):
    c = lax.axis_index("c")
    s = lax.axis_index("s")
    pltpu.sync_copy(zeros_hbm.at[pl.ds(s * NROW, NROW)],
                    acc.at[pl.ds(s * NROW, NROW)])
    pltpu.sync_copy(idx_hbm.at[c, s], idx_v)
    plsc.subcore_barrier()

    @pl.loop(0, NCH)
    def _chunk(ch):
        pltpu.sync_copy(m_hbm.at[c, s, ch], m_v)
        pltpu.sync_copy(m_v, acc.at[idx_v.at[ch]], add=True)

    plsc.subcore_barrier()
    pltpu.sync_copy(acc.at[pl.ds(s * NROW, NROW)],
                    out_hbm.at[c, pl.ds(s * NROW, NROW)])


@functools.cache
def _sc_scatter_kernel():
    return pl.kernel(
        _scatter_body,
        out_type=jax.ShapeDtypeStruct((NC, N, N_H), jnp.float32),
        mesh=plsc.VectorSubcoreMesh(core_axis_name="c", subcore_axis_name="s",
                                    num_cores=NC, num_subcores=NS),
        scratch_types=[
            pltpu.VMEM((C, N_H), jnp.float32),
            pltpu.VMEM((NCH, C), jnp.int32),
            pltpu.VMEM_SHARED((N, N_H), jnp.float32),
        ],
        compiler_params=pltpu.CompilerParams(use_tc_tiling_on_sc=False),
    )


def _einsum_body(a_ref, hg_ref, bb_ref, m_ref):
    hg = hg_ref[...]
    ht = jnp.concatenate([hg] * 8, axis=1)
    p = a_ref[...] * ht
    i = lax.broadcasted_iota(jnp.int32, (64, 8), 0) // 8
    j = lax.broadcasted_iota(jnp.int32, (64, 8), 1)
    r = (i == j).astype(jnp.float32)
    m_ref[...] = jnp.dot(p, r, preferred_element_type=jnp.float32) + bb_ref[...]


def _einsum(a, hg, bb):
    return pl.pallas_call(
        _einsum_body,
        grid=(NEB,),
        in_specs=[
            pl.BlockSpec((EBLK, 64), lambda i: (i, 0)),
            pl.BlockSpec((EBLK, 8), lambda i: (i, 0)),
            pl.BlockSpec((EBLK, 8), lambda i: (i, 0)),
        ],
        out_specs=pl.BlockSpec((EBLK, 8), lambda i: (i, 0)),
        out_shape=jax.ShapeDtypeStruct((E, 8), jnp.float32),
    )(a, hg, bb)


def _gru_body(ms_ref, h_ref, k_ref, rk_ref, b_ref, out_ref):
    m = ms_ref[0] + ms_ref[1]
    h = h_ref[...]
    mx = jnp.dot(m, k_ref[...], preferred_element_type=jnp.float32) + b_ref[0:1]
    mh = jnp.dot(h, rk_ref[...], preferred_element_type=jnp.float32) + b_ref[1:2]
    z = jax.nn.sigmoid(mx[:, 0:8] + mh[:, 0:8])
    r = jax.nn.sigmoid(mx[:, 8:16] + mh[:, 8:16])
    cc = jnp.tanh(mx[:, 16:24] + r * mh[:, 16:24])
    out_ref[...] = z * h + (1.0 - z) * cc


def _gru(ms, h, gru_k, gru_rk, gru_b):
    full = lambda shp: pl.BlockSpec(shp, lambda i: (0,) * len(shp))
    return pl.pallas_call(
        _gru_body,
        grid=(NNB,),
        in_specs=[
            pl.BlockSpec((2, NBLK, 8), lambda i: (0, i, 0)),
            pl.BlockSpec((NBLK, 8), lambda i: (i, 0)),
            full((8, 24)), full((8, 24)), full((2, 24)),
        ],
        out_specs=pl.BlockSpec((NBLK, 8), lambda i: (i, 0)),
        out_shape=jax.ShapeDtypeStruct((N, 8), jnp.float32),
    )(ms, h, gru_k, gru_rk, gru_b)


def _readout_body(h_ref, x_ref, seg_ref, iw1, ib1, iw2, ib2, jw1, jb1, jw2, jb2,
                  fw1, fb1, fw2, fb2, out_ref, nb_ref):
    pid = pl.program_id(0)
    hx = jnp.concatenate([h_ref[...], x_ref[...]], axis=1)
    t1 = jnp.tanh(jnp.dot(hx, iw1[...], preferred_element_type=jnp.float32) + ib1[...])
    rr = jax.nn.sigmoid(jnp.dot(t1, iw2[...], preferred_element_type=jnp.float32) + ib2[...])
    t2 = _selu(jnp.dot(hx, jw1[...], preferred_element_type=jnp.float32) + jb1[...])
    rr = rr * (jnp.dot(t2, jw2[...], preferred_element_type=jnp.float32) + jb2[...])
    seg = seg_ref[0, 0]
    oh = (seg[:, None] == lax.broadcasted_iota(jnp.int32, (NBLK, G), 1))
    oh = oh.astype(jnp.float32)
    partial = lax.dot_general(oh, rr, (((0,), (0,)), ((), ())),
                              preferred_element_type=jnp.float32)

    @pl.when(pid == 0)
    def _init():
        nb_ref[...] = jnp.zeros_like(nb_ref)

    nb_ref[...] += partial

    @pl.when(pid == NNB - 1)
    def _fin():
        nb = nb_ref[...]
        z = _selu(jnp.dot(nb, fw1[...], preferred_element_type=jnp.float32) + fb1[...])
        out_ref[...] = jnp.dot(z, fw2[...], preferred_element_type=jnp.float32) + fb2[...]


def _readout(h, x, seg3, i_w1, i_b1, i_w2, i_b2, j_w1, j_b1, j_w2, j_b2,
             f_w1, f_b1, f_w2, f_b2):
    full = lambda shp: pl.BlockSpec(shp, lambda i: (0,) * len(shp))
    return pl.pallas_call(
        _readout_body,
        grid=(NNB,),
        in_specs=[
            pl.BlockSpec((NBLK, 8), lambda i: (i, 0)),
            pl.BlockSpec((NBLK, 2), lambda i: (i, 0)),
            pl.BlockSpec((1, 1, NBLK), lambda i: (i, 0, 0)),
            full((10, 64)), full((1, 64)), full((64, 64)), full((1, 64)),
            full((10, 64)), full((1, 64)), full((64, 64)), full((1, 64)),
            full((64, 64)), full((1, 64)), full((64, 1)), full((1, 1)),
        ],
        out_specs=pl.BlockSpec((G, 1), lambda i: (0, 0)),
        out_shape=jax.ShapeDtypeStruct((G, 1), jnp.float32),
        scratch_shapes=[pltpu.VMEM((G, 64), jnp.float32)],
    )(h, x, seg3, i_w1, i_b1.reshape(1, 64), i_w2, i_b2.reshape(1, 64),
      j_w1, j_b1.reshape(1, 64), j_w2, j_b2.reshape(1, 64),
      f_w1, f_b1.reshape(1, 64), f_w2, f_b2.reshape(1, 1))


def kernel(x, e, first, second, segment, l_w1, l_b1, l_w2, l_b2, b_w1, b_b1,
           b_w2, b_b2, gru_k, gru_rk, gru_b, i_w1, i_b1, i_w2, i_b2, j_w1,
           j_b1, j_w2, j_b2, f_w1, f_b1, f_w2, f_b2):
    first_r = first.reshape(NC, NS, NCH, C)
    second_r = second.reshape(NC, NS, NCH, C)
    seg3 = segment.reshape(NNB, 1, NBLK)
    zeros_n = jnp.zeros((N, N_H), jnp.float32)

    a, bb = _edge_mlps(e, l_w1, l_b1, l_w2, l_b2, b_w1, b_b1, b_w2, b_b2)

    h = jnp.pad(x, ((0, 0), (0, N_H - 2)))
    for _ in range(3):
        hg = _sc_gather_kernel()(h, first_r).reshape(E, N_H)
        m = _einsum(a, hg, bb)
        ms = _sc_scatter_kernel()(m.reshape(NC, NS, NCH, C, N_H), second_r,
                                  zeros_n)
        h = _gru(ms, h, gru_k, gru_rk, gru_b)

    return _readout(h, x, seg3, i_w1, i_b1, i_w2, i_b2, j_w1, j_b1, j_w2,
                    j_b2, f_w1, f_b1, f_w2, f_b2)


# speedup vs baseline: 2.7578x; 1.2638x over previous
import functools

import jax
import jax.numpy as jnp
from jax import lax
from jax.experimental import pallas as pl
from jax.experimental.pallas import tpu as pltpu
from jax.experimental.pallas import tpu_sc as plsc

N = 50000
E = 800000
N_H = 8
G = 128

NC = 2
NS = 16
NW = NC * NS

CH = 128
NCHW = 196
E_PAD = NW * NCHW * CH
ACC_R = N + 8
NROW = N // NS

C1 = 125
NCH1 = (E // NW) // C1

EBLK = 8000
NEB = E // EBLK
ABLK = 4096
NAB = (E_PAD // 2) // ABLK
NBLK = 2000
NNB = N // NBLK


def _selu(v):
    return 1.0507009873554805 * jnp.where(
        v > 0, v, 1.6732632423543772 * (jnp.exp(jnp.minimum(v, 0.0)) - 1.0))


def _bdot(a, b):
    return jnp.dot(a.astype(jnp.bfloat16), b.astype(jnp.bfloat16),
                   preferred_element_type=jnp.float32)


def _a_mlp_body(e2_ref, lw1, lb1, lw2, lb2, a_ref):
    e2 = e2_ref[...]
    halves = []
    for p in range(2):
        sp = _selu(e2[:, p:p + 1] * lw1[...] + lb1[...])
        halves.append(_bdot(sp, lw2[...])
                      + lb2[...])
    a_ref[...] = jnp.concatenate(halves, axis=1)


def _a_mlp(e2, l_w1, l_b1, l_w2, l_b2):
    full = lambda shp: pl.BlockSpec(shp, lambda i: (0,) * len(shp))
    return pl.pallas_call(
        _a_mlp_body,
        grid=(NAB,),
        in_specs=[
            pl.BlockSpec((ABLK, 2), lambda i: (i, 0)),
            full((1, 64)), full((1, 64)), full((64, 64)), full((1, 64)),
        ],
        out_specs=pl.BlockSpec((ABLK, 128), lambda i: (i, 0)),
        out_shape=jax.ShapeDtypeStruct((E_PAD // 2, 128), jnp.float32),
    )(e2, l_w1, l_b1.reshape(1, 64), l_w2, l_b2.reshape(1, 64))


def _bb_mlp_body(e_ref, bw1, bb1, bw2, bb2, bb_ref):
    s2 = _selu(e_ref[...] * bw1[...] + bb1[...])
    bb_ref[...] = _bdot(s2, bw2[...]) + bb2[...]


def _bb_mlp(e, b_w1, b_b1, b_w2, b_b2):
    full = lambda shp: pl.BlockSpec(shp, lambda i: (0,) * len(shp))
    return pl.pallas_call(
        _bb_mlp_body,
        grid=(NEB,),
        in_specs=[
            pl.BlockSpec((EBLK, 1), lambda i: (i, 0)),
            full((1, 64)), full((1, 64)), full((64, 8)), full((1, 8)),
        ],
        out_specs=pl.BlockSpec((EBLK, 8), lambda i: (i, 0)),
        out_shape=jax.ShapeDtypeStruct((E, 8), jnp.float32),
    )(e, b_w1, b_b1.reshape(1, 64), b_w2, b_b2.reshape(1, 8))


def _scatter_body(m_hbm, idx_hbm, zeros_hbm, out_hbm, m_v, idx_v, acc):
    c = lax.axis_index("c")
    s = lax.axis_index("s")
    pltpu.sync_copy(zeros_hbm.at[pl.ds(s * NROW, NROW)],
                    acc.at[pl.ds(s * NROW, NROW)])
    pltpu.sync_copy(idx_hbm.at[c, s], idx_v)
    plsc.subcore_barrier()

    @pl.loop(0, NCH1)
    def _chunk(ch):
        pltpu.sync_copy(m_hbm.at[c, s, ch], m_v)
        pltpu.sync_copy(m_v, acc.at[idx_v.at[ch]], add=True)

    plsc.subcore_barrier()
    pltpu.sync_copy(acc.at[pl.ds(s * NROW, NROW)],
                    out_hbm.at[c, pl.ds(s * NROW, NROW)])


@functools.cache
def _sc_scatter_kernel():
    return pl.kernel(
        _scatter_body,
        out_type=jax.ShapeDtypeStruct((NC, N, N_H), jnp.float32),
        mesh=plsc.VectorSubcoreMesh(core_axis_name="c", subcore_axis_name="s",
                                    num_cores=NC, num_subcores=NS),
        scratch_types=[
            pltpu.VMEM((C1, N_H), jnp.float32),
            pltpu.VMEM((NCH1, C1), jnp.int32),
            pltpu.VMEM_SHARED((N, N_H), jnp.float32),
        ],
        compiler_params=pltpu.CompilerParams(use_tc_tiling_on_sc=False),
    )


def _fused_body(h_hbm, i1_hbm, i2_hbm, a_hbm, mb_hbm, out_hbm,
                idx1_v, idx2_v, a_b0, a_b1, h_b0, h_b1, m_buf,
                sa0, sa1, sg0, sg1, acc):
    c = lax.axis_index("c")
    s = lax.axis_index("s")
    chb = (c * NS + s) * NCHW
    pltpu.sync_copy(i1_hbm.at[c, s], idx1_v)
    pltpu.sync_copy(i2_hbm.at[c, s], idx2_v)
    pltpu.sync_copy(mb_hbm.at[c, pl.ds(s * NROW, NROW)],
                    acc.at[pl.ds(s * NROW, NROW)])
    plsc.subcore_barrier()

    a_bufs = (a_b0, a_b1)
    h_bufs = (h_b0, h_b1)
    sas = (sa0, sa1)
    sgs = (sg0, sg1)

    def fire(ch, b):
        pltpu.async_copy(a_hbm.at[chb + ch], a_bufs[b], sas[b])
        pltpu.async_copy(h_hbm.at[idx1_v.at[ch]], h_bufs[b], sgs[b])

    fire(0, 0)
    iota16 = lax.iota(jnp.int32, 16)
    col = [jnp.full((16,), k, jnp.int32) for k in range(64)]
    coli = [jnp.full((16,), i, jnp.int32) for i in range(8)]

    @pl.loop(0, NCHW, step=2)
    def _outer(ch0):
        for b in range(2):
            ch = ch0 + b

            @pl.when(ch + 1 < NCHW)
            def _():
                fire(ch + 1, 1 - b)

            pltpu.make_async_copy(a_hbm.at[chb + ch], a_bufs[b], sas[b]).wait()
            pltpu.make_async_copy(h_hbm.at[idx1_v.at[ch]], h_bufs[b],
                                  sgs[b]).wait()
            ab = a_bufs[b]
            hb = h_bufs[b]
            for g in range(8):
                rows = g * 16 + iota16
                hj = [plsc.load_gather(hb, [rows, col[j]]) for j in range(8)]
                for i in range(8):
                    m_i = plsc.load_gather(ab, [rows, col[i * 8]]) * hj[0]
                    for j in range(1, 8):
                        m_i += plsc.load_gather(ab, [rows, col[i * 8 + j]]) * hj[j]
                    plsc.store_scatter(m_buf, [rows, coli[i]], m_i)
            pltpu.sync_copy(m_buf, acc.at[idx2_v.at[ch]], add=True)

    plsc.subcore_barrier()
    pltpu.sync_copy(acc.at[pl.ds(s * NROW, NROW)],
                    out_hbm.at[c, pl.ds(s * NROW, NROW)])


@functools.cache
def _sc_fused_kernel():
    return pl.kernel(
        _fused_body,
        out_type=jax.ShapeDtypeStruct((NC, N, N_H), jnp.float32),
        mesh=plsc.VectorSubcoreMesh(core_axis_name="c", subcore_axis_name="s",
                                    num_cores=NC, num_subcores=NS),
        scratch_types=[
            pltpu.VMEM((NCHW, CH), jnp.int32),
            pltpu.VMEM((NCHW, CH), jnp.int32),
            pltpu.VMEM((CH, 64), jnp.float32),
            pltpu.VMEM((CH, 64), jnp.float32),
            pltpu.VMEM((CH, N_H), jnp.float32),
            pltpu.VMEM((CH, N_H), jnp.float32),
            pltpu.VMEM((CH, N_H), jnp.float32),
            pltpu.SemaphoreType.DMA,
            pltpu.SemaphoreType.DMA,
            pltpu.SemaphoreType.DMA,
            pltpu.SemaphoreType.DMA,
            pltpu.VMEM_SHARED((ACC_R, N_H), jnp.float32),
        ],
        compiler_params=pltpu.CompilerParams(use_tc_tiling_on_sc=False,
                                             needs_layout_passes=False),
    )


def _gru_body(ms_ref, h_ref, k_ref, rk_ref, b_ref, out_ref):
    m = ms_ref[0] + ms_ref[1]
    h = h_ref[...]
    mx = _bdot(m, k_ref[...]) + b_ref[0:1]
    mh = _bdot(h, rk_ref[...]) + b_ref[1:2]
    z = jax.nn.sigmoid(mx[:, 0:8] + mh[:, 0:8])
    r = jax.nn.sigmoid(mx[:, 8:16] + mh[:, 8:16])
    cc = jnp.tanh(mx[:, 16:24] + r * mh[:, 16:24])
    out_ref[...] = z * h + (1.0 - z) * cc


def _gru(ms, h, gru_k, gru_rk, gru_b):
    full = lambda shp: pl.BlockSpec(shp, lambda i: (0,) * len(shp))
    return pl.pallas_call(
        _gru_body,
        grid=(NNB,),
        in_specs=[
            pl.BlockSpec((2, NBLK, 8), lambda i: (0, i, 0)),
            pl.BlockSpec((NBLK, 8), lambda i: (i, 0)),
            full((8, 24)), full((8, 24)), full((2, 24)),
        ],
        out_specs=pl.BlockSpec((NBLK, 8), lambda i: (i, 0)),
        out_shape=jax.ShapeDtypeStruct((N, 8), jnp.float32),
    )(ms, h, gru_k, gru_rk, gru_b)


def _readout_body(h_ref, x_ref, seg_ref, iw1, ib1, iw2, ib2, jw1, jb1, jw2, jb2,
                  fw1, fb1, fw2, fb2, out_ref, nb_ref):
    pid = pl.program_id(0)
    hx = jnp.concatenate([h_ref[...], x_ref[...]], axis=1)
    t1 = jnp.tanh(_bdot(hx, iw1[...]) + ib1[...])
    rr = jax.nn.sigmoid(_bdot(t1, iw2[...]) + ib2[...])
    t2 = _selu(_bdot(hx, jw1[...]) + jb1[...])
    rr = rr * (_bdot(t2, jw2[...]) + jb2[...])
    seg = seg_ref[0, 0]
    oh = (seg[:, None] == lax.broadcasted_iota(jnp.int32, (NBLK, G), 1))
    oh = oh.astype(jnp.float32)
    partial = lax.dot_general(oh, rr, (((0,), (0,)), ((), ())),
                              preferred_element_type=jnp.float32,
                              precision=lax.Precision.HIGHEST)

    @pl.when(pid == 0)
    def _init():
        nb_ref[...] = jnp.zeros_like(nb_ref)

    nb_ref[...] += partial

    @pl.when(pid == NNB - 1)
    def _fin():
        nb = nb_ref[...]
        z = _selu(_bdot(nb, fw1[...]) + fb1[...])
        out_ref[...] = _bdot(z, fw2[...]) + fb2[...]


def _readout(h, x, seg3, i_w1, i_b1, i_w2, i_b2, j_w1, j_b1, j_w2, j_b2,
             f_w1, f_b1, f_w2, f_b2):
    full = lambda shp: pl.BlockSpec(shp, lambda i: (0,) * len(shp))
    return pl.pallas_call(
        _readout_body,
        grid=(NNB,),
        in_specs=[
            pl.BlockSpec((NBLK, 8), lambda i: (i, 0)),
            pl.BlockSpec((NBLK, 2), lambda i: (i, 0)),
            pl.BlockSpec((1, 1, NBLK), lambda i: (i, 0, 0)),
            full((10, 64)), full((1, 64)), full((64, 64)), full((1, 64)),
            full((10, 64)), full((1, 64)), full((64, 64)), full((1, 64)),
            full((64, 64)), full((1, 64)), full((64, 1)), full((1, 1)),
        ],
        out_specs=pl.BlockSpec((G, 1), lambda i: (0, 0)),
        out_shape=jax.ShapeDtypeStruct((G, 1), jnp.float32),
        scratch_shapes=[pltpu.VMEM((G, 64), jnp.float32)],
    )(h, x, seg3, i_w1, i_b1.reshape(1, 64), i_w2, i_b2.reshape(1, 64),
      j_w1, j_b1.reshape(1, 64), j_w2, j_b2.reshape(1, 64),
      f_w1, f_b1.reshape(1, 64), f_w2, f_b2.reshape(1, 1))


def kernel(x, e, first, second, segment, l_w1, l_b1, l_w2, l_b2, b_w1, b_b1,
           b_w2, b_b2, gru_k, gru_rk, gru_b, i_w1, i_b1, i_w2, i_b2, j_w1,
           j_b1, j_w2, j_b2, f_w1, f_b1, f_w2, f_b2):
    first4 = jnp.pad(first, (0, E_PAD - E)).reshape(NC, NS, NCHW, CH)
    second4 = jnp.pad(second, (0, E_PAD - E),
                      constant_values=N).reshape(NC, NS, NCHW, CH)
    second_r1 = second.reshape(NC, NS, NCH1, C1)
    seg3 = segment.reshape(NNB, 1, NBLK)
    zeros_n = jnp.zeros((N, N_H), jnp.float32)
    e2 = jnp.pad(e, ((0, E_PAD - E), (0, 0))).reshape(E_PAD // 2, 2)

    a128 = _a_mlp(e2, l_w1, l_b1, l_w2, l_b2)
    a3 = a128.reshape(E_PAD // CH, CH, 64)
    bb = _bb_mlp(e, b_w1, b_b1, b_w2, b_b2)
    mb = _sc_scatter_kernel()(bb.reshape(NC, NS, NCH1, C1, N_H), second_r1,
                              zeros_n)

    h = jnp.pad(x, ((0, 0), (0, N_H - 2)))
    for _ in range(3):
        ms = _sc_fused_kernel()(h, first4, second4, a3, mb)
        h = _gru(ms, h, gru_k, gru_rk, gru_b)

    return _readout(h, x, seg3, i_w1, i_b1, i_w2, i_b2, j_w1, j_b1, j_w2,
                    j_b2, f_w1, f_b1, f_w2, f_b2)


# speedup vs baseline: 2.8642x; 1.0386x over previous
import functools

import jax
import jax.numpy as jnp
from jax import lax
from jax.experimental import pallas as pl
from jax.experimental.pallas import tpu as pltpu
from jax.experimental.pallas import tpu_sc as plsc

N = 50000
E = 800000
N_H = 8
G = 128

NC = 2
NS = 16
NW = NC * NS

CH = 128
NCHW = 196
E_PAD = NW * NCHW * CH
ACC_R = N + 8
NROW = N // NS

C1 = 125
NCH1 = (E // NW) // C1

EBLK = 8000
NEB = E // EBLK
ABLK = 4096
NAB = (E_PAD // 2) // ABLK
NBLK = 2000
NNB = N // NBLK


def _selu(v):
    return 1.0507009873554805 * jnp.where(
        v > 0, v, 1.6732632423543772 * (jnp.exp(jnp.minimum(v, 0.0)) - 1.0))


def _bdot(a, b):
    return jnp.dot(a.astype(jnp.bfloat16), b.astype(jnp.bfloat16),
                   preferred_element_type=jnp.float32)


def _a_mlp_body(e2_ref, lw1, lb1, lw2, lb2, a_ref):
    e2 = e2_ref[...]
    halves = []
    for p in range(2):
        sp = _selu(e2[:, p:p + 1] * lw1[...] + lb1[...])
        halves.append(_bdot(sp, lw2[...])
                      + lb2[...])
    a_ref[...] = jnp.concatenate(halves, axis=1)


def _a_mlp(e2, l_w1, l_b1, l_w2, l_b2):
    full = lambda shp: pl.BlockSpec(shp, lambda i: (0,) * len(shp))
    return pl.pallas_call(
        _a_mlp_body,
        grid=(NAB,),
        in_specs=[
            pl.BlockSpec((ABLK, 2), lambda i: (i, 0)),
            full((1, 64)), full((1, 64)), full((64, 64)), full((1, 64)),
        ],
        out_specs=pl.BlockSpec((ABLK, 128), lambda i: (i, 0)),
        out_shape=jax.ShapeDtypeStruct((E_PAD // 2, 128), jnp.float32),
    )(e2, l_w1, l_b1.reshape(1, 64), l_w2, l_b2.reshape(1, 64))


def _bb_mlp_body(e_ref, bw1, bb1, bw2, bb2, bb_ref):
    s2 = _selu(e_ref[...] * bw1[...] + bb1[...])
    bb_ref[...] = _bdot(s2, bw2[...]) + bb2[...]


def _bb_mlp(e, b_w1, b_b1, b_w2, b_b2):
    full = lambda shp: pl.BlockSpec(shp, lambda i: (0,) * len(shp))
    return pl.pallas_call(
        _bb_mlp_body,
        grid=(NEB,),
        in_specs=[
            pl.BlockSpec((EBLK, 1), lambda i: (i, 0)),
            full((1, 64)), full((1, 64)), full((64, 8)), full((1, 8)),
        ],
        out_specs=pl.BlockSpec((EBLK, 8), lambda i: (i, 0)),
        out_shape=jax.ShapeDtypeStruct((E, 8), jnp.float32),
    )(e, b_w1, b_b1.reshape(1, 64), b_w2, b_b2.reshape(1, 8))


def _scatter_body(m_hbm, idx_hbm, zeros_hbm, out_hbm, m_v, idx_v, acc):
    c = lax.axis_index("c")
    s = lax.axis_index("s")
    pltpu.sync_copy(zeros_hbm.at[pl.ds(s * NROW, NROW)],
                    acc.at[pl.ds(s * NROW, NROW)])
    pltpu.sync_copy(idx_hbm.at[c, s], idx_v)
    plsc.subcore_barrier()

    @pl.loop(0, NCH1)
    def _chunk(ch):
        pltpu.sync_copy(m_hbm.at[c, s, ch], m_v)
        pltpu.sync_copy(m_v, acc.at[idx_v.at[ch]], add=True)

    plsc.subcore_barrier()
    pltpu.sync_copy(acc.at[pl.ds(s * NROW, NROW)],
                    out_hbm.at[c, pl.ds(s * NROW, NROW)])


@functools.cache
def _sc_scatter_kernel():
    return pl.kernel(
        _scatter_body,
        out_type=jax.ShapeDtypeStruct((NC, N, N_H), jnp.float32),
        mesh=plsc.VectorSubcoreMesh(core_axis_name="c", subcore_axis_name="s",
                                    num_cores=NC, num_subcores=NS),
        scratch_types=[
            pltpu.VMEM((C1, N_H), jnp.float32),
            pltpu.VMEM((NCH1, C1), jnp.int32),
            pltpu.VMEM_SHARED((N, N_H), jnp.float32),
        ],
        compiler_params=pltpu.CompilerParams(use_tc_tiling_on_sc=False),
    )


KBUF = 4


def _fused_body(h_hbm, i1_hbm, i2_hbm, a_hbm, mb_hbm, out_hbm,
                idx1_v, idx2_v, a_bufs, h_bufs, m_buf,
                sas, sgs, acc):
    c = lax.axis_index("c")
    s = lax.axis_index("s")
    chb = (c * NS + s) * NCHW
    pltpu.sync_copy(i1_hbm.at[c, s], idx1_v)
    pltpu.sync_copy(i2_hbm.at[c, s], idx2_v)
    pltpu.sync_copy(mb_hbm.at[c, pl.ds(s * NROW, NROW)],
                    acc.at[pl.ds(s * NROW, NROW)])
    plsc.subcore_barrier()

    def fire(ch, b):
        pltpu.async_copy(a_hbm.at[chb + ch], a_bufs[b], sas[b])
        pltpu.async_copy(h_hbm.at[idx1_v.at[ch]], h_bufs[b], sgs[b])

    for k in range(KBUF - 1):
        fire(k, k)
    iota16 = lax.iota(jnp.int32, 16)
    col = [jnp.full((16,), k, jnp.int32) for k in range(64)]
    coli = [jnp.full((16,), i, jnp.int32) for i in range(8)]

    @pl.loop(0, NCHW, step=KBUF)
    def _outer(ch0):
        for k in range(KBUF):
            ch = ch0 + k

            @pl.when(ch + KBUF - 1 < NCHW)
            def _():
                fire(ch + KBUF - 1, (k + KBUF - 1) % KBUF)

            pltpu.make_async_copy(a_hbm.at[chb + ch], a_bufs[k], sas[k]).wait()
            pltpu.make_async_copy(h_hbm.at[idx1_v.at[ch]], h_bufs[k],
                                  sgs[k]).wait()
            ab = a_bufs[k]
            hb = h_bufs[k]
            for g in range(8):
                rows = g * 16 + iota16
                hj = [plsc.load_gather(hb, [rows, col[j]]) for j in range(8)]
                for i in range(8):
                    m_i = plsc.load_gather(ab, [rows, col[i * 8]]) * hj[0]
                    for j in range(1, 8):
                        m_i += plsc.load_gather(ab, [rows, col[i * 8 + j]]) * hj[j]
                    plsc.store_scatter(m_buf, [rows, coli[i]], m_i)
            pltpu.sync_copy(m_buf, acc.at[idx2_v.at[ch]], add=True)

    plsc.subcore_barrier()
    pltpu.sync_copy(acc.at[pl.ds(s * NROW, NROW)],
                    out_hbm.at[c, pl.ds(s * NROW, NROW)])


@functools.cache
def _sc_fused_kernel():
    return pl.kernel(
        _fused_body,
        out_type=jax.ShapeDtypeStruct((NC, N, N_H), jnp.float32),
        mesh=plsc.VectorSubcoreMesh(core_axis_name="c", subcore_axis_name="s",
                                    num_cores=NC, num_subcores=NS),
        scratch_types=[
            pltpu.VMEM((NCHW, CH), jnp.int32),
            pltpu.VMEM((NCHW, CH), jnp.int32),
            [pltpu.VMEM((CH, 64), jnp.float32)] * KBUF,
            [pltpu.VMEM((CH, N_H), jnp.float32)] * KBUF,
            pltpu.VMEM((CH, N_H), jnp.float32),
            [pltpu.SemaphoreType.DMA] * KBUF,
            [pltpu.SemaphoreType.DMA] * KBUF,
            pltpu.VMEM_SHARED((ACC_R, N_H), jnp.float32),
        ],
        compiler_params=pltpu.CompilerParams(use_tc_tiling_on_sc=False,
                                             needs_layout_passes=False),
    )


def _gru_body(ms_ref, h_ref, k_ref, rk_ref, b_ref, out_ref):
    m = ms_ref[0] + ms_ref[1]
    h = h_ref[...]
    mx = _bdot(m, k_ref[...]) + b_ref[0:1]
    mh = _bdot(h, rk_ref[...]) + b_ref[1:2]
    z = jax.nn.sigmoid(mx[:, 0:8] + mh[:, 0:8])
    r = jax.nn.sigmoid(mx[:, 8:16] + mh[:, 8:16])
    cc = jnp.tanh(mx[:, 16:24] + r * mh[:, 16:24])
    out_ref[...] = z * h + (1.0 - z) * cc


def _gru(ms, h, gru_k, gru_rk, gru_b):
    full = lambda shp: pl.BlockSpec(shp, lambda i: (0,) * len(shp))
    return pl.pallas_call(
        _gru_body,
        grid=(NNB,),
        in_specs=[
            pl.BlockSpec((2, NBLK, 8), lambda i: (0, i, 0)),
            pl.BlockSpec((NBLK, 8), lambda i: (i, 0)),
            full((8, 24)), full((8, 24)), full((2, 24)),
        ],
        out_specs=pl.BlockSpec((NBLK, 8), lambda i: (i, 0)),
        out_shape=jax.ShapeDtypeStruct((N, 8), jnp.float32),
    )(ms, h, gru_k, gru_rk, gru_b)


def _readout_body(h_ref, x_ref, seg_ref, iw1, ib1, iw2, ib2, jw1, jb1, jw2, jb2,
                  fw1, fb1, fw2, fb2, out_ref, nb_ref):
    pid = pl.program_id(0)
    hx = jnp.concatenate([h_ref[...], x_ref[...]], axis=1)
    t1 = jnp.tanh(_bdot(hx, iw1[...]) + ib1[...])
    rr = jax.nn.sigmoid(_bdot(t1, iw2[...]) + ib2[...])
    t2 = _selu(_bdot(hx, jw1[...]) + jb1[...])
    rr = rr * (_bdot(t2, jw2[...]) + jb2[...])
    seg = seg_ref[0, 0]
    oh = (seg[:, None] == lax.broadcasted_iota(jnp.int32, (NBLK, G), 1))
    oh = oh.astype(jnp.float32)
    partial = lax.dot_general(oh, rr, (((0,), (0,)), ((), ())),
                              preferred_element_type=jnp.float32,
                              precision=lax.Precision.HIGHEST)

    @pl.when(pid == 0)
    def _init():
        nb_ref[...] = jnp.zeros_like(nb_ref)

    nb_ref[...] += partial

    @pl.when(pid == NNB - 1)
    def _fin():
        nb = nb_ref[...]
        z = _selu(_bdot(nb, fw1[...]) + fb1[...])
        out_ref[...] = _bdot(z, fw2[...]) + fb2[...]


def _readout(h, x, seg3, i_w1, i_b1, i_w2, i_b2, j_w1, j_b1, j_w2, j_b2,
             f_w1, f_b1, f_w2, f_b2):
    full = lambda shp: pl.BlockSpec(shp, lambda i: (0,) * len(shp))
    return pl.pallas_call(
        _readout_body,
        grid=(NNB,),
        in_specs=[
            pl.BlockSpec((NBLK, 8), lambda i: (i, 0)),
            pl.BlockSpec((NBLK, 2), lambda i: (i, 0)),
            pl.BlockSpec((1, 1, NBLK), lambda i: (i, 0, 0)),
            full((10, 64)), full((1, 64)), full((64, 64)), full((1, 64)),
            full((10, 64)), full((1, 64)), full((64, 64)), full((1, 64)),
            full((64, 64)), full((1, 64)), full((64, 1)), full((1, 1)),
        ],
        out_specs=pl.BlockSpec((G, 1), lambda i: (0, 0)),
        out_shape=jax.ShapeDtypeStruct((G, 1), jnp.float32),
        scratch_shapes=[pltpu.VMEM((G, 64), jnp.float32)],
    )(h, x, seg3, i_w1, i_b1.reshape(1, 64), i_w2, i_b2.reshape(1, 64),
      j_w1, j_b1.reshape(1, 64), j_w2, j_b2.reshape(1, 64),
      f_w1, f_b1.reshape(1, 64), f_w2, f_b2.reshape(1, 1))


def kernel(x, e, first, second, segment, l_w1, l_b1, l_w2, l_b2, b_w1, b_b1,
           b_w2, b_b2, gru_k, gru_rk, gru_b, i_w1, i_b1, i_w2, i_b2, j_w1,
           j_b1, j_w2, j_b2, f_w1, f_b1, f_w2, f_b2):
    first4 = jnp.pad(first, (0, E_PAD - E)).reshape(NC, NS, NCHW, CH)
    second4 = jnp.pad(second, (0, E_PAD - E),
                      constant_values=N).reshape(NC, NS, NCHW, CH)
    second_r1 = second.reshape(NC, NS, NCH1, C1)
    seg3 = segment.reshape(NNB, 1, NBLK)
    zeros_n = jnp.zeros((N, N_H), jnp.float32)
    e2 = jnp.pad(e, ((0, E_PAD - E), (0, 0))).reshape(E_PAD // 2, 2)

    a128 = _a_mlp(e2, l_w1, l_b1, l_w2, l_b2)
    a3 = a128.reshape(E_PAD // CH, CH, 64)
    bb = _bb_mlp(e, b_w1, b_b1, b_w2, b_b2)
    mb = _sc_scatter_kernel()(bb.reshape(NC, NS, NCH1, C1, N_H), second_r1,
                              zeros_n)

    h = jnp.pad(x, ((0, 0), (0, N_H - 2)))
    for _ in range(3):
        ms = _sc_fused_kernel()(h, first4, second4, a3, mb)
        h = _gru(ms, h, gru_k, gru_rk, gru_b)

    return _readout(h, x, seg3, i_w1, i_b1, i_w2, i_b2, j_w1, j_b1, j_w2,
                    j_b2, f_w1, f_b1, f_w2, f_b2)


# speedup vs baseline: 2.8675x; 1.0011x over previous
import functools

import jax
import jax.numpy as jnp
from jax import lax
from jax.experimental import pallas as pl
from jax.experimental.pallas import tpu as pltpu
from jax.experimental.pallas import tpu_sc as plsc

N = 50000
E = 800000
N_H = 8
G = 128

NC = 2
NS = 16
NW = NC * NS

CH = 128
NCHW = 196
E_PAD = NW * NCHW * CH
ACC_R = N + 8
NROW = N // NS

C1 = 125
NCH1 = (E // NW) // C1

EBLK = 8000
NEB = E // EBLK
ABLK = 4096
NAB = (E_PAD // 2) // ABLK
NBLK = 2000
NNB = N // NBLK


def _selu(v):
    return 1.0507009873554805 * jnp.where(
        v > 0, v, 1.6732632423543772 * (jnp.exp(jnp.minimum(v, 0.0)) - 1.0))


def _bdot(a, b):
    return jnp.dot(a.astype(jnp.bfloat16), b.astype(jnp.bfloat16),
                   preferred_element_type=jnp.float32)


def _a_mlp_body(e2_ref, lw1, lb1, lw2, lb2, a_ref):
    e2 = e2_ref[...]
    halves = []
    for p in range(2):
        sp = _selu(e2[:, p:p + 1] * lw1[...] + lb1[...])
        halves.append(_bdot(sp, lw2[...])
                      + lb2[...])
    a_ref[...] = jnp.concatenate(halves, axis=1)


def _a_mlp(e2, l_w1, l_b1, l_w2, l_b2):
    full = lambda shp: pl.BlockSpec(shp, lambda i: (0,) * len(shp))
    return pl.pallas_call(
        _a_mlp_body,
        grid=(NAB,),
        in_specs=[
            pl.BlockSpec((ABLK, 2), lambda i: (i, 0)),
            full((1, 64)), full((1, 64)), full((64, 64)), full((1, 64)),
        ],
        out_specs=pl.BlockSpec((ABLK, 128), lambda i: (i, 0)),
        out_shape=jax.ShapeDtypeStruct((E_PAD // 2, 128), jnp.float32),
    )(e2, l_w1, l_b1.reshape(1, 64), l_w2, l_b2.reshape(1, 64))


def _bb_mlp_body(e_ref, bw1, bb1, bw2, bb2, bb_ref):
    s2 = _selu(e_ref[...] * bw1[...] + bb1[...])
    bb_ref[...] = _bdot(s2, bw2[...]) + bb2[...]


def _bb_mlp(e, b_w1, b_b1, b_w2, b_b2):
    full = lambda shp: pl.BlockSpec(shp, lambda i: (0,) * len(shp))
    return pl.pallas_call(
        _bb_mlp_body,
        grid=(NEB,),
        in_specs=[
            pl.BlockSpec((EBLK, 1), lambda i: (i, 0)),
            full((1, 64)), full((1, 64)), full((64, 8)), full((1, 8)),
        ],
        out_specs=pl.BlockSpec((EBLK, 8), lambda i: (i, 0)),
        out_shape=jax.ShapeDtypeStruct((E, 8), jnp.float32),
    )(e, b_w1, b_b1.reshape(1, 64), b_w2, b_b2.reshape(1, 8))


def _scatter_body(m_hbm, idx_hbm, zeros_hbm, out_hbm, m_v, idx_v, acc):
    c = lax.axis_index("c")
    s = lax.axis_index("s")
    pltpu.sync_copy(zeros_hbm.at[pl.ds(s * NROW, NROW)],
                    acc.at[pl.ds(s * NROW, NROW)])
    pltpu.sync_copy(idx_hbm.at[c, s], idx_v)
    plsc.subcore_barrier()

    @pl.loop(0, NCH1)
    def _chunk(ch):
        pltpu.sync_copy(m_hbm.at[c, s, ch], m_v)
        pltpu.sync_copy(m_v, acc.at[idx_v.at[ch]], add=True)

    plsc.subcore_barrier()
    pltpu.sync_copy(acc.at[pl.ds(s * NROW, NROW)],
                    out_hbm.at[c, pl.ds(s * NROW, NROW)])


@functools.cache
def _sc_scatter_kernel():
    return pl.kernel(
        _scatter_body,
        out_type=jax.ShapeDtypeStruct((NC, N, N_H), jnp.float32),
        mesh=plsc.VectorSubcoreMesh(core_axis_name="c", subcore_axis_name="s",
                                    num_cores=NC, num_subcores=NS),
        scratch_types=[
            pltpu.VMEM((C1, N_H), jnp.float32),
            pltpu.VMEM((NCH1, C1), jnp.int32),
            pltpu.VMEM_SHARED((N, N_H), jnp.float32),
        ],
        compiler_params=pltpu.CompilerParams(use_tc_tiling_on_sc=False),
    )


KBUF = 4


def _fused_body(h_hbm, i1_hbm, i2_hbm, a_hbm, mb_hbm, out_hbm,
                idx1_v, idx2_v, a_bufs, h_bufs, m_buf,
                sas, sgs, acc):
    c = lax.axis_index("c")
    s = lax.axis_index("s")
    chb = (c * NS + s) * NCHW
    pltpu.sync_copy(i1_hbm.at[c, s], idx1_v)
    pltpu.sync_copy(i2_hbm.at[c, s], idx2_v)
    pltpu.sync_copy(mb_hbm.at[c, pl.ds(s * NROW, NROW)],
                    acc.at[pl.ds(s * NROW, NROW)])
    plsc.subcore_barrier()

    def fire(ch, b):
        pltpu.async_copy(a_hbm.at[chb + ch], a_bufs[b], sas[b])
        pltpu.async_copy(h_hbm.at[idx1_v.at[ch]], h_bufs[b], sgs[b])

    for k in range(KBUF - 1):
        fire(k, k)
    iota16 = lax.iota(jnp.int32, 16)
    colj = [jnp.full((16,), j, jnp.int32) for j in range(8)]

    @pl.loop(0, NCHW, step=KBUF)
    def _outer(ch0):
        for k in range(KBUF):
            ch = ch0 + k

            @pl.when(ch + KBUF - 1 < NCHW)
            def _():
                fire(ch + KBUF - 1, (k + KBUF - 1) % KBUF)

            pltpu.make_async_copy(a_hbm.at[chb + ch], a_bufs[k], sas[k]).wait()
            pltpu.make_async_copy(h_hbm.at[idx1_v.at[ch]], h_bufs[k],
                                  sgs[k]).wait()
            ab = a_bufs[k]
            hb = h_bufs[k]
            for g in range(8):
                rows = g * 16 + iota16
                base = rows * 64
                hj = [plsc.load_gather(hb, [rows, colj[j]]) for j in range(8)]
                for i in range(8):
                    m_i = plsc.load_gather(ab, [base + i * 8]) * hj[0]
                    for j in range(1, 8):
                        m_i += plsc.load_gather(ab, [base + (i * 8 + j)]) * hj[j]
                    plsc.store_scatter(m_buf, [rows, colj[i]], m_i)
            pltpu.sync_copy(m_buf, acc.at[idx2_v.at[ch]], add=True)

    plsc.subcore_barrier()
    pltpu.sync_copy(acc.at[pl.ds(s * NROW, NROW)],
                    out_hbm.at[c, pl.ds(s * NROW, NROW)])


@functools.cache
def _sc_fused_kernel():
    return pl.kernel(
        _fused_body,
        out_type=jax.ShapeDtypeStruct((NC, N, N_H), jnp.float32),
        mesh=plsc.VectorSubcoreMesh(core_axis_name="c", subcore_axis_name="s",
                                    num_cores=NC, num_subcores=NS),
        scratch_types=[
            pltpu.VMEM((NCHW, CH), jnp.int32),
            pltpu.VMEM((NCHW, CH), jnp.int32),
            [pltpu.VMEM((CH * 64,), jnp.float32)] * KBUF,
            [pltpu.VMEM((CH, N_H), jnp.float32)] * KBUF,
            pltpu.VMEM((CH, N_H), jnp.float32),
            [pltpu.SemaphoreType.DMA] * KBUF,
            [pltpu.SemaphoreType.DMA] * KBUF,
            pltpu.VMEM_SHARED((ACC_R, N_H), jnp.float32),
        ],
        compiler_params=pltpu.CompilerParams(use_tc_tiling_on_sc=False,
                                             needs_layout_passes=False),
    )


def _gru_body(ms_ref, h_ref, k_ref, rk_ref, b_ref, out_ref):
    m = ms_ref[0] + ms_ref[1]
    h = h_ref[...]
    mx = _bdot(m, k_ref[...]) + b_ref[0:1]
    mh = _bdot(h, rk_ref[...]) + b_ref[1:2]
    z = jax.nn.sigmoid(mx[:, 0:8] + mh[:, 0:8])
    r = jax.nn.sigmoid(mx[:, 8:16] + mh[:, 8:16])
    cc = jnp.tanh(mx[:, 16:24] + r * mh[:, 16:24])
    out_ref[...] = z * h + (1.0 - z) * cc


def _gru(ms, h, gru_k, gru_rk, gru_b):
    full = lambda shp: pl.BlockSpec(shp, lambda i: (0,) * len(shp))
    return pl.pallas_call(
        _gru_body,
        grid=(NNB,),
        in_specs=[
            pl.BlockSpec((2, NBLK, 8), lambda i: (0, i, 0)),
            pl.BlockSpec((NBLK, 8), lambda i: (i, 0)),
            full((8, 24)), full((8, 24)), full((2, 24)),
        ],
        out_specs=pl.BlockSpec((NBLK, 8), lambda i: (i, 0)),
        out_shape=jax.ShapeDtypeStruct((N, 8), jnp.float32),
    )(ms, h, gru_k, gru_rk, gru_b)


def _readout_body(h_ref, x_ref, seg_ref, iw1, ib1, iw2, ib2, jw1, jb1, jw2, jb2,
                  fw1, fb1, fw2, fb2, out_ref, nb_ref):
    pid = pl.program_id(0)
    hx = jnp.concatenate([h_ref[...], x_ref[...]], axis=1)
    t1 = jnp.tanh(_bdot(hx, iw1[...]) + ib1[...])
    rr = jax.nn.sigmoid(_bdot(t1, iw2[...]) + ib2[...])
    t2 = _selu(_bdot(hx, jw1[...]) + jb1[...])
    rr = rr * (_bdot(t2, jw2[...]) + jb2[...])
    seg = seg_ref[0, 0]
    oh = (seg[:, None] == lax.broadcasted_iota(jnp.int32, (NBLK, G), 1))
    oh = oh.astype(jnp.float32)
    partial = lax.dot_general(oh, rr, (((0,), (0,)), ((), ())),
                              preferred_element_type=jnp.float32,
                              precision=lax.Precision.HIGHEST)

    @pl.when(pid == 0)
    def _init():
        nb_ref[...] = jnp.zeros_like(nb_ref)

    nb_ref[...] += partial

    @pl.when(pid == NNB - 1)
    def _fin():
        nb = nb_ref[...]
        z = _selu(_bdot(nb, fw1[...]) + fb1[...])
        out_ref[...] = _bdot(z, fw2[...]) + fb2[...]


def _readout(h, x, seg3, i_w1, i_b1, i_w2, i_b2, j_w1, j_b1, j_w2, j_b2,
             f_w1, f_b1, f_w2, f_b2):
    full = lambda shp: pl.BlockSpec(shp, lambda i: (0,) * len(shp))
    return pl.pallas_call(
        _readout_body,
        grid=(NNB,),
        in_specs=[
            pl.BlockSpec((NBLK, 8), lambda i: (i, 0)),
            pl.BlockSpec((NBLK, 2), lambda i: (i, 0)),
            pl.BlockSpec((1, 1, NBLK), lambda i: (i, 0, 0)),
            full((10, 64)), full((1, 64)), full((64, 64)), full((1, 64)),
            full((10, 64)), full((1, 64)), full((64, 64)), full((1, 64)),
            full((64, 64)), full((1, 64)), full((64, 1)), full((1, 1)),
        ],
        out_specs=pl.BlockSpec((G, 1), lambda i: (0, 0)),
        out_shape=jax.ShapeDtypeStruct((G, 1), jnp.float32),
        scratch_shapes=[pltpu.VMEM((G, 64), jnp.float32)],
    )(h, x, seg3, i_w1, i_b1.reshape(1, 64), i_w2, i_b2.reshape(1, 64),
      j_w1, j_b1.reshape(1, 64), j_w2, j_b2.reshape(1, 64),
      f_w1, f_b1.reshape(1, 64), f_w2, f_b2.reshape(1, 1))


def kernel(x, e, first, second, segment, l_w1, l_b1, l_w2, l_b2, b_w1, b_b1,
           b_w2, b_b2, gru_k, gru_rk, gru_b, i_w1, i_b1, i_w2, i_b2, j_w1,
           j_b1, j_w2, j_b2, f_w1, f_b1, f_w2, f_b2):
    first4 = jnp.pad(first, (0, E_PAD - E)).reshape(NC, NS, NCHW, CH)
    second4 = jnp.pad(second, (0, E_PAD - E),
                      constant_values=N).reshape(NC, NS, NCHW, CH)
    second_r1 = second.reshape(NC, NS, NCH1, C1)
    seg3 = segment.reshape(NNB, 1, NBLK)
    zeros_n = jnp.zeros((N, N_H), jnp.float32)
    e2 = jnp.pad(e, ((0, E_PAD - E), (0, 0))).reshape(E_PAD // 2, 2)

    a128 = _a_mlp(e2, l_w1, l_b1, l_w2, l_b2)
    a3 = a128.reshape(E_PAD // CH, CH * 64)
    bb = _bb_mlp(e, b_w1, b_b1, b_w2, b_b2)
    mb = _sc_scatter_kernel()(bb.reshape(NC, NS, NCH1, C1, N_H), second_r1,
                              zeros_n)

    h = jnp.pad(x, ((0, 0), (0, N_H - 2)))
    for _ in range(3):
        ms = _sc_fused_kernel()(h, first4, second4, a3, mb)
        h = _gru(ms, h, gru_k, gru_rk, gru_b)

    return _readout(h, x, seg3, i_w1, i_b1, i_w2, i_b2, j_w1, j_b1, j_w2,
                    j_b2, f_w1, f_b1, f_w2, f_b2)
